# Initial kernel scaffold; baseline (speedup 1.0000x reference)
import jax
import jax.numpy as jnp
from jax.experimental import pallas as pl


def kernel(x, edge_index1, edge_index2, W1, W2, W3, W4):
    raise NotImplementedError("write your pallas kernel here")


# speedup vs baseline: 22.8952x; 22.8952x over previous
import functools

import jax
import jax.numpy as jnp
from jax import lax
from jax.experimental import pallas as pl
from jax.experimental.pallas import tpu as pltpu
from jax.experimental.pallas import tpu_sc as plsc

N = 10000
E = 320000
D = 128
H1, H2, H3, H4 = 64, 32, 32, 16

NC, NS = 2, 16
NT = NC * NS
CH = 80
CPT = E // NT // CH
HCPT = E // NS // CH
NP = 10240
ROWS = NP // NS

_MESH = plsc.VectorSubcoreMesh(
    core_axis_name="c", subcore_axis_name="s", num_cores=NC, num_subcores=NS)

_f32 = jnp.float32
_SC_PARAMS = pltpu.CompilerParams(use_tc_tiling_on_sc=False)


def _hist_body(eidx, ones_hbm, zeros_hbm, douts, dins,
               acc_do, acc_di, ones_v, src_v, dst_v):
    cid = lax.axis_index("c")
    sid = lax.axis_index("s")
    r0 = sid * ROWS
    pltpu.sync_copy(ones_hbm, ones_v)
    pltpu.sync_copy(zeros_hbm, acc_do.at[pl.ds(r0, ROWS)])
    pltpu.sync_copy(zeros_hbm, acc_di.at[pl.ds(r0, ROWS)])
    pltpu.sync_copy(eidx.at[cid, 0, sid], src_v)
    pltpu.sync_copy(eidx.at[cid, 1, sid], dst_v)
    plsc.subcore_barrier()

    @pl.loop(0, HCPT)
    def _(c):
        pltpu.sync_copy(ones_v, acc_do.at[src_v.at[c]], add=True)
        pltpu.sync_copy(ones_v, acc_di.at[dst_v.at[c]], add=True)

    plsc.subcore_barrier()
    pltpu.sync_copy(acc_do.at[pl.ds(r0, ROWS)], douts.at[cid, pl.ds(r0, ROWS)])
    pltpu.sync_copy(acc_di.at[pl.ds(r0, ROWS)], dins.at[cid, pl.ds(r0, ROWS)])


_hist = pl.kernel(
    _hist_body,
    out_type=(jax.ShapeDtypeStruct((NC, NP, 16), _f32),
              jax.ShapeDtypeStruct((NC, NP, 16), _f32)),
    mesh=_MESH,
    compiler_params=_SC_PARAMS,
    scratch_types=[
        pltpu.VMEM_SHARED((NP, 16), _f32),
        pltpu.VMEM_SHARED((NP, 16), _f32),
        pltpu.VMEM((CH, 16), _f32),
        pltpu.VMEM((HCPT, CH), jnp.int32),
        pltpu.VMEM((HCPT, CH), jnp.int32),
    ],
)


def _seg_body(t_hbm, srcr, dstr, zeros_hbm, out_hbm,
              acc, src_v, dst_v, rows_v, sem):
    cid = lax.axis_index("c")
    sid = lax.axis_index("s")
    w = cid * NS + sid
    r0 = sid * ROWS
    pltpu.sync_copy(zeros_hbm, acc.at[pl.ds(r0, ROWS)])
    pltpu.sync_copy(srcr.at[w], src_v)
    pltpu.sync_copy(dstr.at[w], dst_v)
    plsc.subcore_barrier()

    pltpu.async_copy(t_hbm.at[src_v.at[0]], rows_v.at[0], sem).wait()

    @pl.loop(0, CPT - 1)
    def _(c):
        b = lax.rem(c, 2)
        nb = lax.rem(c + 1, 2)
        cp = pltpu.async_copy(t_hbm.at[src_v.at[c + 1]], rows_v.at[nb], sem)
        pltpu.sync_copy(rows_v.at[b], acc.at[dst_v.at[c]], add=True)
        cp.wait()

    lb = lax.rem(CPT - 1, 2)
    pltpu.sync_copy(rows_v.at[lb], acc.at[dst_v.at[CPT - 1]], add=True)

    plsc.subcore_barrier()
    pltpu.sync_copy(acc.at[pl.ds(r0, ROWS)], out_hbm.at[cid, pl.ds(r0, ROWS)])


def _make_seg(h):
    return pl.kernel(
        _seg_body,
        out_type=jax.ShapeDtypeStruct((NC, NP, h), _f32),
        mesh=_MESH,
        compiler_params=_SC_PARAMS,
        scratch_types=[
            pltpu.VMEM_SHARED((NP, h), _f32),
            pltpu.VMEM((CPT, CH), jnp.int32),
            pltpu.VMEM((CPT, CH), jnp.int32),
            pltpu.VMEM((2, CH, h), _f32),
            pltpu.SemaphoreType.DMA,
        ],
    )


_seg = {h: _make_seg(h) for h in (H1, H2, H4)}


BT = 2000


def _l1_body(x_ref, hdo_ref, w_ref, o_ref):
    p = lax.rsqrt(hdo_ref[:, 0:1] + 1.0)
    o_ref[...] = jnp.dot(x_ref[...], w_ref[...],
                         preferred_element_type=_f32) * p


def _mid_body(parts_ref, hq_ref, hp_ref, w_ref, o_ref):
    a = parts_ref[0] + parts_ref[1]
    q = lax.rsqrt(hq_ref[:, 0:1] + 1.0)
    h = jnp.maximum(a * q, 0.0)
    p = lax.rsqrt(hp_ref[:, 0:1] + 1.0)
    o_ref[...] = jnp.dot(h, w_ref[...], preferred_element_type=_f32) * p


def _fin_body(parts_ref, hq_ref, o_ref):
    q = lax.rsqrt(hq_ref[:, 0:1] + 1.0)
    o_ref[...] = (parts_ref[0] + parts_ref[1]) * q


def _tc_l1(x, hdo, w):
    din, dout = w.shape
    return pl.pallas_call(
        _l1_body,
        grid=(N // BT,),
        in_specs=[
            pl.BlockSpec((BT, din), lambda i: (i, 0)),
            pl.BlockSpec((BT, 16), lambda i: (i, 0)),
            pl.BlockSpec((din, dout), lambda i: (0, 0)),
        ],
        out_specs=pl.BlockSpec((BT, dout), lambda i: (i, 0)),
        out_shape=jax.ShapeDtypeStruct((N, dout), _f32),
    )(x, hdo, w)


def _tc_mid(parts, hq, hp, w):
    din, dout = w.shape
    return pl.pallas_call(
        _mid_body,
        grid=(N // BT,),
        in_specs=[
            pl.BlockSpec((NC, BT, din), lambda i: (0, i, 0)),
            pl.BlockSpec((BT, 16), lambda i: (i, 0)),
            pl.BlockSpec((BT, 16), lambda i: (i, 0)),
            pl.BlockSpec((din, dout), lambda i: (0, 0)),
        ],
        out_specs=pl.BlockSpec((BT, dout), lambda i: (i, 0)),
        out_shape=jax.ShapeDtypeStruct((N, dout), _f32),
    )(parts, hq, hp, w)


def _tc_fin(parts, hq):
    dout = parts.shape[-1]
    return pl.pallas_call(
        _fin_body,
        grid=(N // BT,),
        in_specs=[
            pl.BlockSpec((NC, BT, dout), lambda i: (0, i, 0)),
            pl.BlockSpec((BT, 16), lambda i: (i, 0)),
        ],
        out_specs=pl.BlockSpec((BT, dout), lambda i: (i, 0)),
        out_shape=jax.ShapeDtypeStruct((N, dout), _f32),
    )(parts, hq)


def kernel(x, edge_index1, edge_index2, W1, W2, W3, W4):
    eidx = jnp.stack([edge_index1, edge_index2]).reshape(2, 2, NS, HCPT, CH)
    ones16 = jnp.ones((CH, 16), _f32)
    z16 = jnp.zeros((ROWS, 16), _f32)
    douts, dins = _hist(eidx, ones16, z16)
    hdo1, hdo2 = douts[0], douts[1]
    hdi1, hdi2 = dins[0], dins[1]

    src1 = edge_index1[0].reshape(NT, CPT, CH)
    dst1 = edge_index1[1].reshape(NT, CPT, CH)
    src2 = edge_index2[0].reshape(NT, CPT, CH)
    dst2 = edge_index2[1].reshape(NT, CPT, CH)
    z64 = jnp.zeros((ROWS, H1), _f32)
    z32 = jnp.zeros((ROWS, H2), _f32)
    zh4 = jnp.zeros((ROWS, H4), _f32)

    t0 = _tc_l1(x, hdo1, W1)
    a1 = _seg[H1](t0, src1, dst1, z64)
    t1 = _tc_mid(a1, hdi1, hdo1, W2)
    a2 = _seg[H2](t1, src1, dst1, z32)
    t2 = _tc_mid(a2, hdi1, hdo2, W3)
    a3 = _seg[H3](t2, src2, dst2, z32)
    t3 = _tc_mid(a3, hdi2, hdo2, W4)
    a4 = _seg[H4](t3, src2, dst2, zh4)
    return _tc_fin(a4, hdi2)


# speedup vs baseline: 43.0322x; 1.8795x over previous
import functools

import jax
import jax.numpy as jnp
from jax import lax
from jax.experimental import pallas as pl
from jax.experimental.pallas import tpu as pltpu
from jax.experimental.pallas import tpu_sc as plsc

N = 10000
E = 320000
D = 128
H1, H2, H3, H4 = 64, 32, 32, 16

NC, NS = 2, 16
NT = NC * NS
MCH = 128
NCHK = E // MCH
SEG_MAIN = NCHK // NT
HIST_MAIN = NCHK // NS
XTRA = NCHK - NT * SEG_MAIN
NP = 10240
ROWS = NP // NS

_MESH = plsc.VectorSubcoreMesh(
    core_axis_name="c", subcore_axis_name="s", num_cores=NC, num_subcores=NS)

_f32 = jnp.float32
_SC_PARAMS = pltpu.CompilerParams(use_tc_tiling_on_sc=False)


def _hist_body(e1, e2, ones_hbm, zeros_hbm, douts, dins,
               acc_do, acc_di, ones_v, src_v, dst_v, xsrc_v, xdst_v, hsem):
    cid = lax.axis_index("c")
    sid = lax.axis_index("s")
    r0 = sid * ROWS
    pltpu.sync_copy(ones_hbm, ones_v)
    pltpu.sync_copy(zeros_hbm, acc_do.at[pl.ds(r0, ROWS)])
    pltpu.sync_copy(zeros_hbm, acc_di.at[pl.ds(r0, ROWS)])
    plsc.subcore_barrier()

    def run(e):
        pltpu.sync_copy(e.at[0, pl.ds(sid * HIST_MAIN, HIST_MAIN)], src_v)
        pltpu.sync_copy(e.at[1, pl.ds(sid * HIST_MAIN, HIST_MAIN)], dst_v)

        for c in range(3):
            pltpu.async_copy(ones_v, acc_do.at[src_v.at[c]], hsem, add=True)
            pltpu.async_copy(ones_v, acc_di.at[dst_v.at[c]], hsem, add=True)

        @pl.loop(3, HIST_MAIN)
        def _(c):
            pltpu.async_copy(ones_v, acc_do.at[src_v.at[c]], hsem, add=True)
            pltpu.async_copy(ones_v, acc_di.at[dst_v.at[c]], hsem, add=True)
            pltpu.make_async_copy(ones_hbm, ones_v, hsem).wait()
            pltpu.make_async_copy(ones_hbm, ones_v, hsem).wait()

        for _c in range(3):
            pltpu.make_async_copy(ones_hbm, ones_v, hsem).wait()
            pltpu.make_async_copy(ones_hbm, ones_v, hsem).wait()

        @pl.when(sid < XTRA)
        def _():
            pltpu.sync_copy(e.at[0, NS * HIST_MAIN + sid], xsrc_v)
            pltpu.sync_copy(e.at[1, NS * HIST_MAIN + sid], xdst_v)
            d1 = pltpu.async_copy(ones_v, acc_do.at[xsrc_v], hsem, add=True)
            d2 = pltpu.async_copy(ones_v, acc_di.at[xdst_v], hsem, add=True)
            d1.wait()
            d2.wait()

    @pl.when(cid == 0)
    def _():
        run(e1)

    @pl.when(cid == 1)
    def _():
        run(e2)

    plsc.subcore_barrier()
    pltpu.sync_copy(acc_do.at[pl.ds(r0, ROWS)], douts.at[cid, pl.ds(r0, ROWS)])
    pltpu.sync_copy(acc_di.at[pl.ds(r0, ROWS)], dins.at[cid, pl.ds(r0, ROWS)])


_hist = pl.kernel(
    _hist_body,
    out_type=(jax.ShapeDtypeStruct((NC, NP, 16), _f32),
              jax.ShapeDtypeStruct((NC, NP, 16), _f32)),
    mesh=_MESH,
    compiler_params=_SC_PARAMS,
    scratch_types=[
        pltpu.VMEM_SHARED((NP, 16), _f32),
        pltpu.VMEM_SHARED((NP, 16), _f32),
        pltpu.VMEM((MCH, 16), _f32),
        pltpu.VMEM((HIST_MAIN, MCH), jnp.int32),
        pltpu.VMEM((HIST_MAIN, MCH), jnp.int32),
        pltpu.VMEM((MCH,), jnp.int32),
        pltpu.VMEM((MCH,), jnp.int32),
        pltpu.SemaphoreType.DMA,
    ],
)


def _seg_body(t_hbm, e, zeros_hbm, out_hbm,
              acc, src_v, dst_v, xsrc_v, xdst_v, rows_v,
              g0, g1, g2, g3):
    cid = lax.axis_index("c")
    sid = lax.axis_index("s")
    w = cid * NS + sid
    r0 = sid * ROWS
    pltpu.sync_copy(zeros_hbm, acc.at[pl.ds(r0, ROWS)])
    pltpu.sync_copy(e.at[0, pl.ds(w * SEG_MAIN, SEG_MAIN)], src_v)
    pltpu.sync_copy(e.at[1, pl.ds(w * SEG_MAIN, SEG_MAIN)], dst_v)
    plsc.subcore_barrier()

    gs = (g0, g1, g2, g3)
    dummy = t_hbm.at[pl.ds(0, MCH)]
    for j in range(3):
        pltpu.async_copy(t_hbm.at[src_v.at[j]], rows_v.at[j], gs[j])

    @pl.loop(0, SEG_MAIN // 4 - 1)
    def _(g):
        c0 = g * 4
        for j in range(4):
            jn = (j + 3) % 4
            pltpu.make_async_copy(dummy, rows_v.at[j], gs[j]).wait()
            pltpu.async_copy(t_hbm.at[src_v.at[c0 + j + 3]], rows_v.at[jn],
                             gs[jn])
            pltpu.sync_copy(rows_v.at[j], acc.at[dst_v.at[c0 + j]], add=True)

    c0 = (SEG_MAIN // 4 - 1) * 4
    for j in range(6):
        c = c0 + j
        b = c % 4
        pltpu.make_async_copy(dummy, rows_v.at[b], gs[b]).wait()
        if c + 3 < SEG_MAIN:
            bn = (c + 3) % 4
            pltpu.async_copy(t_hbm.at[src_v.at[c + 3]], rows_v.at[bn], gs[bn])
        pltpu.sync_copy(rows_v.at[b], acc.at[dst_v.at[c]], add=True)

    @pl.when(w < XTRA)
    def _():
        pltpu.sync_copy(e.at[0, NT * SEG_MAIN + w], xsrc_v)
        pltpu.sync_copy(e.at[1, NT * SEG_MAIN + w], xdst_v)
        pltpu.async_copy(t_hbm.at[xsrc_v], rows_v.at[0], gs[0]).wait()
        pltpu.sync_copy(rows_v.at[0], acc.at[xdst_v], add=True)

    plsc.subcore_barrier()
    pltpu.sync_copy(acc.at[pl.ds(r0, ROWS)], out_hbm.at[cid, pl.ds(r0, ROWS)])


def _make_seg(h):
    return pl.kernel(
        _seg_body,
        out_type=jax.ShapeDtypeStruct((NC, NP, h), _f32),
        mesh=_MESH,
        compiler_params=_SC_PARAMS,
        scratch_types=[
            pltpu.VMEM_SHARED((NP, h), _f32),
            pltpu.VMEM((SEG_MAIN, MCH), jnp.int32),
            pltpu.VMEM((SEG_MAIN, MCH), jnp.int32),
            pltpu.VMEM((MCH,), jnp.int32),
            pltpu.VMEM((MCH,), jnp.int32),
            pltpu.VMEM((4, MCH, h), _f32),
            pltpu.SemaphoreType.DMA,
            pltpu.SemaphoreType.DMA,
            pltpu.SemaphoreType.DMA,
            pltpu.SemaphoreType.DMA,
        ],
    )


_seg = {h: _make_seg(h) for h in (H1, H2, H4)}


BT = 2000


def _l1_body(x_ref, hq_ref, w_ref, o_ref):
    p = lax.rsqrt(hq_ref[0, :, 0:1] + 1.0)
    o_ref[...] = jnp.dot(x_ref[...], w_ref[...],
                         preferred_element_type=_f32) * p


def _mid_body(parts_ref, hq_ref, hp_ref, w_ref, o_ref):
    a = parts_ref[0] + parts_ref[1]
    q = lax.rsqrt(hq_ref[0, :, 0:1] + 1.0)
    h = jnp.maximum(a * q, 0.0)
    p = lax.rsqrt(hp_ref[0, :, 0:1] + 1.0)
    o_ref[...] = jnp.dot(h, w_ref[...], preferred_element_type=_f32) * p


def _fin_body(parts_ref, hq_ref, o_ref):
    q = lax.rsqrt(hq_ref[0, :, 0:1] + 1.0)
    o_ref[...] = (parts_ref[0] + parts_ref[1]) * q


def _hist_spec(set_idx):
    return pl.BlockSpec((1, BT, 16), lambda i, s=set_idx: (s, i, 0))


def _tc_l1(x, douts, w):
    din, dout = w.shape
    return pl.pallas_call(
        _l1_body,
        grid=(N // BT,),
        in_specs=[
            pl.BlockSpec((BT, din), lambda i: (i, 0)),
            _hist_spec(0),
            pl.BlockSpec((din, dout), lambda i: (0, 0)),
        ],
        out_specs=pl.BlockSpec((BT, dout), lambda i: (i, 0)),
        out_shape=jax.ShapeDtypeStruct((N, dout), _f32),
    )(x, douts, w)


def _tc_mid(parts, dins_a, douts_a, w, qset, pset):
    din, dout = w.shape
    return pl.pallas_call(
        _mid_body,
        grid=(N // BT,),
        in_specs=[
            pl.BlockSpec((NC, BT, din), lambda i: (0, i, 0)),
            _hist_spec(qset),
            _hist_spec(pset),
            pl.BlockSpec((din, dout), lambda i: (0, 0)),
        ],
        out_specs=pl.BlockSpec((BT, dout), lambda i: (i, 0)),
        out_shape=jax.ShapeDtypeStruct((N, dout), _f32),
    )(parts, dins_a, douts_a, w)


def _tc_fin(parts, dins_a, qset):
    dout = parts.shape[-1]
    return pl.pallas_call(
        _fin_body,
        grid=(N // BT,),
        in_specs=[
            pl.BlockSpec((NC, BT, dout), lambda i: (0, i, 0)),
            _hist_spec(qset),
        ],
        out_specs=pl.BlockSpec((BT, dout), lambda i: (i, 0)),
        out_shape=jax.ShapeDtypeStruct((N, dout), _f32),
    )(parts, dins_a)


def kernel(x, edge_index1, edge_index2, W1, W2, W3, W4):
    e1 = edge_index1.reshape(2, NCHK, MCH)
    e2 = edge_index2.reshape(2, NCHK, MCH)
    ones16 = jnp.ones((MCH, 16), _f32)
    z16 = jnp.zeros((ROWS, 16), _f32)
    douts, dins = _hist(e1, e2, ones16, z16)
    z64 = jnp.zeros((ROWS, H1), _f32)
    z32 = jnp.zeros((ROWS, H2), _f32)
    zh4 = jnp.zeros((ROWS, H4), _f32)

    t0 = _tc_l1(x, douts, W1)
    a1 = _seg[H1](t0, e1, z64)
    t1 = _tc_mid(a1, dins, douts, W2, 0, 0)
    a2 = _seg[H2](t1, e1, z32)
    t2 = _tc_mid(a2, dins, douts, W3, 0, 1)
    a3 = _seg[H3](t2, e2, z32)
    t3 = _tc_mid(a3, dins, douts, W4, 1, 1)
    a4 = _seg[H4](t3, e2, zh4)
    return _tc_fin(a4, dins, 1)


# speedup vs baseline: 43.6901x; 1.0153x over previous
import functools

import jax
import jax.numpy as jnp
from jax import lax
from jax.experimental import pallas as pl
from jax.experimental.pallas import tpu as pltpu
from jax.experimental.pallas import tpu_sc as plsc

N = 10000
E = 320000
D = 128
H1, H2, H3, H4 = 64, 32, 32, 16

NC, NS = 2, 16
NT = NC * NS
MCH = 128
NCHK = E // MCH
SEG_MAIN = NCHK // NT
HIST_MAIN = NCHK // NS
XTRA = NCHK - NT * SEG_MAIN
NP = 10240
ROWS = NP // NS

_MESH = plsc.VectorSubcoreMesh(
    core_axis_name="c", subcore_axis_name="s", num_cores=NC, num_subcores=NS)

_f32 = jnp.float32
_SC_PARAMS = pltpu.CompilerParams(use_tc_tiling_on_sc=False)


def _hist_body(e1, e2, ones_hbm, zeros_hbm, douts, dins,
               acc_do, acc_di, ones_v, src_v, dst_v, xsrc_v, xdst_v, hsem):
    cid = lax.axis_index("c")
    sid = lax.axis_index("s")
    r0 = sid * ROWS
    pltpu.sync_copy(ones_hbm, ones_v)
    pltpu.sync_copy(zeros_hbm, acc_do.at[pl.ds(r0, ROWS)])
    pltpu.sync_copy(zeros_hbm, acc_di.at[pl.ds(r0, ROWS)])
    plsc.subcore_barrier()

    def run(e):
        pltpu.sync_copy(e.at[0, pl.ds(sid * HIST_MAIN, HIST_MAIN)], src_v)
        pltpu.sync_copy(e.at[1, pl.ds(sid * HIST_MAIN, HIST_MAIN)], dst_v)

        for c in range(3):
            pltpu.async_copy(ones_v, acc_do.at[src_v.at[c]], hsem, add=True)
            pltpu.async_copy(ones_v, acc_di.at[dst_v.at[c]], hsem, add=True)

        @pl.loop(3, HIST_MAIN)
        def _(c):
            pltpu.async_copy(ones_v, acc_do.at[src_v.at[c]], hsem, add=True)
            pltpu.async_copy(ones_v, acc_di.at[dst_v.at[c]], hsem, add=True)
            pltpu.make_async_copy(ones_hbm, ones_v, hsem).wait()
            pltpu.make_async_copy(ones_hbm, ones_v, hsem).wait()

        for _c in range(3):
            pltpu.make_async_copy(ones_hbm, ones_v, hsem).wait()
            pltpu.make_async_copy(ones_hbm, ones_v, hsem).wait()

        @pl.when(sid < XTRA)
        def _():
            pltpu.sync_copy(e.at[0, NS * HIST_MAIN + sid], xsrc_v)
            pltpu.sync_copy(e.at[1, NS * HIST_MAIN + sid], xdst_v)
            d1 = pltpu.async_copy(ones_v, acc_do.at[xsrc_v], hsem, add=True)
            d2 = pltpu.async_copy(ones_v, acc_di.at[xdst_v], hsem, add=True)
            d1.wait()
            d2.wait()

    @pl.when(cid == 0)
    def _():
        run(e1)

    @pl.when(cid == 1)
    def _():
        run(e2)

    plsc.subcore_barrier()
    pltpu.sync_copy(acc_do.at[pl.ds(r0, ROWS)], douts.at[cid, pl.ds(r0, ROWS)])
    pltpu.sync_copy(acc_di.at[pl.ds(r0, ROWS)], dins.at[cid, pl.ds(r0, ROWS)])


_hist = pl.kernel(
    _hist_body,
    out_type=(jax.ShapeDtypeStruct((NC, NP, 16), _f32),
              jax.ShapeDtypeStruct((NC, NP, 16), _f32)),
    mesh=_MESH,
    compiler_params=_SC_PARAMS,
    scratch_types=[
        pltpu.VMEM_SHARED((NP, 16), _f32),
        pltpu.VMEM_SHARED((NP, 16), _f32),
        pltpu.VMEM((MCH, 16), _f32),
        pltpu.VMEM((HIST_MAIN, MCH), jnp.int32),
        pltpu.VMEM((HIST_MAIN, MCH), jnp.int32),
        pltpu.VMEM((MCH,), jnp.int32),
        pltpu.VMEM((MCH,), jnp.int32),
        pltpu.SemaphoreType.DMA,
    ],
)


def _seg_body(t_hbm, e, zeros_hbm, out_hbm,
              acc, src_v, dst_v, xsrc_v, xdst_v, rows_v,
              g0, g1, g2, g3):
    cid = lax.axis_index("c")
    sid = lax.axis_index("s")
    w = cid * NS + sid
    r0 = sid * ROWS
    pltpu.sync_copy(zeros_hbm, acc.at[pl.ds(r0, ROWS)])
    pltpu.sync_copy(e.at[0, pl.ds(w * SEG_MAIN, SEG_MAIN)], src_v)
    pltpu.sync_copy(e.at[1, pl.ds(w * SEG_MAIN, SEG_MAIN)], dst_v)
    plsc.subcore_barrier()

    gs = (g0, g1, g2, g3)
    dummy = t_hbm.at[pl.ds(0, MCH)]
    for j in range(3):
        pltpu.async_copy(t_hbm.at[src_v.at[j]], rows_v.at[j], gs[j])

    @pl.loop(0, SEG_MAIN // 4 - 1)
    def _(g):
        c0 = g * 4
        for j in range(4):
            jn = (j + 3) % 4
            pltpu.make_async_copy(dummy, rows_v.at[j], gs[j]).wait()
            pltpu.async_copy(t_hbm.at[src_v.at[c0 + j + 3]], rows_v.at[jn],
                             gs[jn])
            pltpu.sync_copy(rows_v.at[j], acc.at[dst_v.at[c0 + j]], add=True)

    c0 = (SEG_MAIN // 4 - 1) * 4
    for j in range(6):
        c = c0 + j
        b = c % 4
        pltpu.make_async_copy(dummy, rows_v.at[b], gs[b]).wait()
        if c + 3 < SEG_MAIN:
            bn = (c + 3) % 4
            pltpu.async_copy(t_hbm.at[src_v.at[c + 3]], rows_v.at[bn], gs[bn])
        pltpu.sync_copy(rows_v.at[b], acc.at[dst_v.at[c]], add=True)

    @pl.when(w < XTRA)
    def _():
        pltpu.sync_copy(e.at[0, NT * SEG_MAIN + w], xsrc_v)
        pltpu.sync_copy(e.at[1, NT * SEG_MAIN + w], xdst_v)
        pltpu.async_copy(t_hbm.at[xsrc_v], rows_v.at[0], gs[0]).wait()
        pltpu.sync_copy(rows_v.at[0], acc.at[xdst_v], add=True)

    plsc.subcore_barrier()
    pltpu.sync_copy(acc.at[pl.ds(r0, ROWS)], out_hbm.at[cid, pl.ds(r0, ROWS)])


def _make_seg(h):
    return pl.kernel(
        _seg_body,
        out_type=jax.ShapeDtypeStruct((NC, NP, h), _f32),
        mesh=_MESH,
        compiler_params=_SC_PARAMS,
        scratch_types=[
            pltpu.VMEM_SHARED((NP, h), _f32),
            pltpu.VMEM((SEG_MAIN, MCH), jnp.int32),
            pltpu.VMEM((SEG_MAIN, MCH), jnp.int32),
            pltpu.VMEM((MCH,), jnp.int32),
            pltpu.VMEM((MCH,), jnp.int32),
            pltpu.VMEM((4, MCH, h), _f32),
            pltpu.SemaphoreType.DMA,
            pltpu.SemaphoreType.DMA,
            pltpu.SemaphoreType.DMA,
            pltpu.SemaphoreType.DMA,
        ],
    )


_seg = {h: _make_seg(h) for h in (H1, H2, H4)}


BT = 5000


def _l1_body(x_ref, hq_ref, w_ref, o_ref):
    p = lax.rsqrt(hq_ref[0, :, 0:1] + 1.0)
    o_ref[...] = jnp.dot(x_ref[...], w_ref[...],
                         preferred_element_type=_f32) * p


def _mid_body(parts_ref, hq_ref, hp_ref, w_ref, o_ref):
    a = parts_ref[0] + parts_ref[1]
    q = lax.rsqrt(hq_ref[0, :, 0:1] + 1.0)
    h = jnp.maximum(a * q, 0.0)
    p = lax.rsqrt(hp_ref[0, :, 0:1] + 1.0)
    o_ref[...] = jnp.dot(h, w_ref[...], preferred_element_type=_f32) * p


def _fin_body(parts_ref, hq_ref, o_ref):
    q = lax.rsqrt(hq_ref[0, :, 0:1] + 1.0)
    o_ref[...] = (parts_ref[0] + parts_ref[1]) * q


def _hist_spec(set_idx):
    return pl.BlockSpec((1, BT, 16), lambda i, s=set_idx: (s, i, 0))


def _tc_l1(x, douts, w):
    din, dout = w.shape
    return pl.pallas_call(
        _l1_body,
        grid=(N // BT,),
        in_specs=[
            pl.BlockSpec((BT, din), lambda i: (i, 0)),
            _hist_spec(0),
            pl.BlockSpec((din, dout), lambda i: (0, 0)),
        ],
        out_specs=pl.BlockSpec((BT, dout), lambda i: (i, 0)),
        out_shape=jax.ShapeDtypeStruct((N, dout), _f32),
    )(x, douts, w)


def _tc_mid(parts, dins_a, douts_a, w, qset, pset):
    din, dout = w.shape
    return pl.pallas_call(
        _mid_body,
        grid=(N // BT,),
        in_specs=[
            pl.BlockSpec((NC, BT, din), lambda i: (0, i, 0)),
            _hist_spec(qset),
            _hist_spec(pset),
            pl.BlockSpec((din, dout), lambda i: (0, 0)),
        ],
        out_specs=pl.BlockSpec((BT, dout), lambda i: (i, 0)),
        out_shape=jax.ShapeDtypeStruct((N, dout), _f32),
    )(parts, dins_a, douts_a, w)


def _tc_fin(parts, dins_a, qset):
    dout = parts.shape[-1]
    return pl.pallas_call(
        _fin_body,
        grid=(N // BT,),
        in_specs=[
            pl.BlockSpec((NC, BT, dout), lambda i: (0, i, 0)),
            _hist_spec(qset),
        ],
        out_specs=pl.BlockSpec((BT, dout), lambda i: (i, 0)),
        out_shape=jax.ShapeDtypeStruct((N, dout), _f32),
    )(parts, dins_a)


def kernel(x, edge_index1, edge_index2, W1, W2, W3, W4):
    e1 = edge_index1.reshape(2, NCHK, MCH)
    e2 = edge_index2.reshape(2, NCHK, MCH)
    ones16 = jnp.ones((MCH, 16), _f32)
    z16 = jnp.zeros((ROWS, 16), _f32)
    douts, dins = _hist(e1, e2, ones16, z16)
    z64 = jnp.zeros((ROWS, H1), _f32)
    z32 = jnp.zeros((ROWS, H2), _f32)
    zh4 = jnp.zeros((ROWS, H4), _f32)

    t0 = _tc_l1(x, douts, W1)
    a1 = _seg[H1](t0, e1, z64)
    t1 = _tc_mid(a1, dins, douts, W2, 0, 0)
    a2 = _seg[H2](t1, e1, z32)
    t2 = _tc_mid(a2, dins, douts, W3, 0, 1)
    a3 = _seg[H3](t2, e2, z32)
    t3 = _tc_mid(a3, dins, douts, W4, 1, 1)
    a4 = _seg[H4](t3, e2, zh4)
    return _tc_fin(a4, dins, 1)


# speedup vs baseline: 44.4516x; 1.0174x over previous
import functools

import jax
import jax.numpy as jnp
from jax import lax
from jax.experimental import pallas as pl
from jax.experimental.pallas import tpu as pltpu
from jax.experimental.pallas import tpu_sc as plsc

N = 10000
E = 320000
D = 128
H1, H2, H3, H4 = 64, 32, 32, 16

NC, NS = 2, 16
NT = NC * NS
MCH = 128
NCHK = E // MCH
SEG_MAIN = NCHK // NT
HIST_MAIN = NCHK // NS
XTRA = NCHK - NT * SEG_MAIN
NP = 10240
ROWS = NP // NS

_MESH = plsc.VectorSubcoreMesh(
    core_axis_name="c", subcore_axis_name="s", num_cores=NC, num_subcores=NS)

_f32 = jnp.float32
_SC_PARAMS = pltpu.CompilerParams(use_tc_tiling_on_sc=False)


def _hist_body(e1, e2, ones_hbm, zeros_hbm, douts, dins,
               acc_do, acc_di, ones_v, src_v, dst_v, xsrc_v, xdst_v, hsem):
    cid = lax.axis_index("c")
    sid = lax.axis_index("s")
    r0 = sid * ROWS
    pltpu.sync_copy(ones_hbm, ones_v)
    pltpu.sync_copy(zeros_hbm, acc_do.at[pl.ds(r0, ROWS)])
    pltpu.sync_copy(zeros_hbm, acc_di.at[pl.ds(r0, ROWS)])
    plsc.subcore_barrier()

    def run(e):
        pltpu.sync_copy(e.at[pl.ds(sid * HIST_MAIN, HIST_MAIN), 0], src_v)
        pltpu.sync_copy(e.at[pl.ds(sid * HIST_MAIN, HIST_MAIN), 1], dst_v)

        for c in range(3):
            pltpu.async_copy(ones_v, acc_do.at[src_v.at[c]], hsem, add=True)
            pltpu.async_copy(ones_v, acc_di.at[dst_v.at[c]], hsem, add=True)

        @pl.loop(3, HIST_MAIN)
        def _(c):
            pltpu.async_copy(ones_v, acc_do.at[src_v.at[c]], hsem, add=True)
            pltpu.async_copy(ones_v, acc_di.at[dst_v.at[c]], hsem, add=True)
            pltpu.make_async_copy(ones_hbm, ones_v, hsem).wait()
            pltpu.make_async_copy(ones_hbm, ones_v, hsem).wait()

        for _c in range(3):
            pltpu.make_async_copy(ones_hbm, ones_v, hsem).wait()
            pltpu.make_async_copy(ones_hbm, ones_v, hsem).wait()

        @pl.when(sid < XTRA)
        def _():
            pltpu.sync_copy(e.at[NS * HIST_MAIN + sid, 0], xsrc_v)
            pltpu.sync_copy(e.at[NS * HIST_MAIN + sid, 1], xdst_v)
            d1 = pltpu.async_copy(ones_v, acc_do.at[xsrc_v], hsem, add=True)
            d2 = pltpu.async_copy(ones_v, acc_di.at[xdst_v], hsem, add=True)
            d1.wait()
            d2.wait()

    @pl.when(cid == 0)
    def _():
        run(e1)

    @pl.when(cid == 1)
    def _():
        run(e2)

    plsc.subcore_barrier()
    pltpu.sync_copy(acc_do.at[pl.ds(r0, ROWS)], douts.at[cid, pl.ds(r0, ROWS)])
    pltpu.sync_copy(acc_di.at[pl.ds(r0, ROWS)], dins.at[cid, pl.ds(r0, ROWS)])


_hist = pl.kernel(
    _hist_body,
    out_type=(jax.ShapeDtypeStruct((NC, NP, 16), _f32),
              jax.ShapeDtypeStruct((NC, NP, 16), _f32)),
    mesh=_MESH,
    compiler_params=_SC_PARAMS,
    scratch_types=[
        pltpu.VMEM_SHARED((NP, 16), _f32),
        pltpu.VMEM_SHARED((NP, 16), _f32),
        pltpu.VMEM((MCH, 16), _f32),
        pltpu.VMEM((HIST_MAIN, MCH), jnp.int32),
        pltpu.VMEM((HIST_MAIN, MCH), jnp.int32),
        pltpu.VMEM((MCH,), jnp.int32),
        pltpu.VMEM((MCH,), jnp.int32),
        pltpu.SemaphoreType.DMA,
    ],
)


def _seg_body(t_hbm, e, zeros_hbm, out_hbm,
              acc, src_v, dst_v, xsrc_v, xdst_v, rows_v,
              g0, g1, g2, g3):
    cid = lax.axis_index("c")
    sid = lax.axis_index("s")
    w = cid * NS + sid
    r0 = sid * ROWS
    pltpu.sync_copy(zeros_hbm, acc.at[pl.ds(r0, ROWS)])
    pltpu.sync_copy(e.at[pl.ds(w * SEG_MAIN, SEG_MAIN), 0], src_v)
    pltpu.sync_copy(e.at[pl.ds(w * SEG_MAIN, SEG_MAIN), 1], dst_v)
    plsc.subcore_barrier()

    gs = (g0, g1, g2, g3)
    dummy = t_hbm.at[pl.ds(0, MCH)]
    for j in range(3):
        pltpu.async_copy(t_hbm.at[src_v.at[j]], rows_v.at[j], gs[j])

    @pl.loop(0, SEG_MAIN // 4 - 1)
    def _(g):
        c0 = g * 4
        for j in range(4):
            jn = (j + 3) % 4
            pltpu.make_async_copy(dummy, rows_v.at[j], gs[j]).wait()
            pltpu.async_copy(t_hbm.at[src_v.at[c0 + j + 3]], rows_v.at[jn],
                             gs[jn])
            pltpu.sync_copy(rows_v.at[j], acc.at[dst_v.at[c0 + j]], add=True)

    c0 = (SEG_MAIN // 4 - 1) * 4
    for j in range(6):
        c = c0 + j
        b = c % 4
        pltpu.make_async_copy(dummy, rows_v.at[b], gs[b]).wait()
        if c + 3 < SEG_MAIN:
            bn = (c + 3) % 4
            pltpu.async_copy(t_hbm.at[src_v.at[c + 3]], rows_v.at[bn], gs[bn])
        pltpu.sync_copy(rows_v.at[b], acc.at[dst_v.at[c]], add=True)

    @pl.when(w < XTRA)
    def _():
        pltpu.sync_copy(e.at[NT * SEG_MAIN + w, 0], xsrc_v)
        pltpu.sync_copy(e.at[NT * SEG_MAIN + w, 1], xdst_v)
        pltpu.async_copy(t_hbm.at[xsrc_v], rows_v.at[0], gs[0]).wait()
        pltpu.sync_copy(rows_v.at[0], acc.at[xdst_v], add=True)

    plsc.subcore_barrier()
    pltpu.sync_copy(acc.at[pl.ds(r0, ROWS)], out_hbm.at[cid, pl.ds(r0, ROWS)])


def _make_seg(h):
    return pl.kernel(
        _seg_body,
        out_type=jax.ShapeDtypeStruct((NC, NP, h), _f32),
        mesh=_MESH,
        compiler_params=_SC_PARAMS,
        scratch_types=[
            pltpu.VMEM_SHARED((NP, h), _f32),
            pltpu.VMEM((SEG_MAIN, MCH), jnp.int32),
            pltpu.VMEM((SEG_MAIN, MCH), jnp.int32),
            pltpu.VMEM((MCH,), jnp.int32),
            pltpu.VMEM((MCH,), jnp.int32),
            pltpu.VMEM((4, MCH, h), _f32),
            pltpu.SemaphoreType.DMA,
            pltpu.SemaphoreType.DMA,
            pltpu.SemaphoreType.DMA,
            pltpu.SemaphoreType.DMA,
        ],
    )


_seg = {h: _make_seg(h) for h in (H1, H2, H4)}


BT = 5000


def _l1_body(x_ref, hq_ref, w_ref, o_ref):
    p = lax.rsqrt(hq_ref[0, :, 0:1] + 1.0)
    o_ref[...] = jnp.dot(x_ref[...], w_ref[...],
                         preferred_element_type=_f32) * p


def _mid_body(parts_ref, hq_ref, hp_ref, w_ref, o_ref):
    a = parts_ref[0] + parts_ref[1]
    q = lax.rsqrt(hq_ref[0, :, 0:1] + 1.0)
    h = jnp.maximum(a * q, 0.0)
    p = lax.rsqrt(hp_ref[0, :, 0:1] + 1.0)
    o_ref[...] = jnp.dot(h, w_ref[...], preferred_element_type=_f32) * p


def _fin_body(parts_ref, hq_ref, o_ref):
    q = lax.rsqrt(hq_ref[0, :, 0:1] + 1.0)
    o_ref[...] = (parts_ref[0] + parts_ref[1]) * q


def _hist_spec(set_idx):
    return pl.BlockSpec((1, BT, 16), lambda i, s=set_idx: (s, i, 0))


def _tc_l1(x, douts, w):
    din, dout = w.shape
    return pl.pallas_call(
        _l1_body,
        grid=(N // BT,),
        in_specs=[
            pl.BlockSpec((BT, din), lambda i: (i, 0)),
            _hist_spec(0),
            pl.BlockSpec((din, dout), lambda i: (0, 0)),
        ],
        out_specs=pl.BlockSpec((BT, dout), lambda i: (i, 0)),
        out_shape=jax.ShapeDtypeStruct((N, dout), _f32),
    )(x, douts, w)


def _tc_mid(parts, dins_a, douts_a, w, qset, pset):
    din, dout = w.shape
    return pl.pallas_call(
        _mid_body,
        grid=(N // BT,),
        in_specs=[
            pl.BlockSpec((NC, BT, din), lambda i: (0, i, 0)),
            _hist_spec(qset),
            _hist_spec(pset),
            pl.BlockSpec((din, dout), lambda i: (0, 0)),
        ],
        out_specs=pl.BlockSpec((BT, dout), lambda i: (i, 0)),
        out_shape=jax.ShapeDtypeStruct((N, dout), _f32),
    )(parts, dins_a, douts_a, w)


def _tc_fin(parts, dins_a, qset):
    dout = parts.shape[-1]
    return pl.pallas_call(
        _fin_body,
        grid=(N // BT,),
        in_specs=[
            pl.BlockSpec((NC, BT, dout), lambda i: (0, i, 0)),
            _hist_spec(qset),
        ],
        out_specs=pl.BlockSpec((BT, dout), lambda i: (i, 0)),
        out_shape=jax.ShapeDtypeStruct((N, dout), _f32),
    )(parts, dins_a)


def kernel(x, edge_index1, edge_index2, W1, W2, W3, W4):
    e1 = jnp.transpose(edge_index1.reshape(2, NCHK, MCH), (1, 0, 2))
    e2 = jnp.transpose(edge_index2.reshape(2, NCHK, MCH), (1, 0, 2))
    ones16 = jnp.ones((MCH, 16), _f32)
    z16 = jnp.zeros((ROWS, 16), _f32)
    douts, dins = _hist(e1, e2, ones16, z16)
    z64 = jnp.zeros((ROWS, H1), _f32)
    z32 = jnp.zeros((ROWS, H2), _f32)
    zh4 = jnp.zeros((ROWS, H4), _f32)

    t0 = _tc_l1(x, douts, W1)
    a1 = _seg[H1](t0, e1, z64)
    t1 = _tc_mid(a1, dins, douts, W2, 0, 0)
    a2 = _seg[H2](t1, e1, z32)
    t2 = _tc_mid(a2, dins, douts, W3, 0, 1)
    a3 = _seg[H3](t2, e2, z32)
    t3 = _tc_mid(a3, dins, douts, W4, 1, 1)
    a4 = _seg[H4](t3, e2, zh4)
    return _tc_fin(a4, dins, 1)


# speedup vs baseline: 46.3259x; 1.0422x over previous
import functools

import jax
import jax.numpy as jnp
from jax import lax
from jax.experimental import pallas as pl
from jax.experimental.pallas import tpu as pltpu
from jax.experimental.pallas import tpu_sc as plsc

N = 10000
E = 320000
D = 128
H1, H2, H3, H4 = 64, 32, 32, 16

NC, NS = 2, 16
NT = NC * NS
MCH = 128
NCHK = E // MCH
SEG_MAIN = NCHK // NT
HIST_MAIN = NCHK // NS
XTRA = NCHK - NT * SEG_MAIN
NP = 10240
ROWS = NP // NS

_MESH = plsc.VectorSubcoreMesh(
    core_axis_name="c", subcore_axis_name="s", num_cores=NC, num_subcores=NS)

_f32 = jnp.float32
_SC_PARAMS = pltpu.CompilerParams(use_tc_tiling_on_sc=False)


def _hist_body(e1, e2, ones_hbm, zeros_hbm, douts, dins,
               acc_do, acc_di, ones_v, src_v, dst_v, xsrc_v, xdst_v, hsem):
    cid = lax.axis_index("c")
    sid = lax.axis_index("s")
    r0 = sid * ROWS
    pltpu.sync_copy(ones_hbm, ones_v)
    pltpu.sync_copy(zeros_hbm, acc_do.at[pl.ds(r0, ROWS)])
    pltpu.sync_copy(zeros_hbm, acc_di.at[pl.ds(r0, ROWS)])
    plsc.subcore_barrier()

    def run(e):
        pltpu.sync_copy(e.at[pl.ds(sid * HIST_MAIN, HIST_MAIN), 0], src_v)
        pltpu.sync_copy(e.at[pl.ds(sid * HIST_MAIN, HIST_MAIN), 1], dst_v)

        for c in range(3):
            pltpu.async_copy(ones_v, acc_do.at[src_v.at[c]], hsem, add=True)
            pltpu.async_copy(ones_v, acc_di.at[dst_v.at[c]], hsem, add=True)

        @pl.loop(3, HIST_MAIN)
        def _(c):
            pltpu.async_copy(ones_v, acc_do.at[src_v.at[c]], hsem, add=True)
            pltpu.async_copy(ones_v, acc_di.at[dst_v.at[c]], hsem, add=True)
            pltpu.make_async_copy(ones_hbm, ones_v, hsem).wait()
            pltpu.make_async_copy(ones_hbm, ones_v, hsem).wait()

        for _c in range(3):
            pltpu.make_async_copy(ones_hbm, ones_v, hsem).wait()
            pltpu.make_async_copy(ones_hbm, ones_v, hsem).wait()

        @pl.when(sid < XTRA)
        def _():
            pltpu.sync_copy(e.at[NS * HIST_MAIN + sid, 0], xsrc_v)
            pltpu.sync_copy(e.at[NS * HIST_MAIN + sid, 1], xdst_v)
            d1 = pltpu.async_copy(ones_v, acc_do.at[xsrc_v], hsem, add=True)
            d2 = pltpu.async_copy(ones_v, acc_di.at[xdst_v], hsem, add=True)
            d1.wait()
            d2.wait()

    @pl.when(cid == 0)
    def _():
        run(e1)

    @pl.when(cid == 1)
    def _():
        run(e2)

    plsc.subcore_barrier()
    pltpu.sync_copy(acc_do.at[pl.ds(r0, ROWS)], douts.at[cid, pl.ds(r0, ROWS)])
    pltpu.sync_copy(acc_di.at[pl.ds(r0, ROWS)], dins.at[cid, pl.ds(r0, ROWS)])


_hist = pl.kernel(
    _hist_body,
    out_type=(jax.ShapeDtypeStruct((NC, NP, 16), _f32),
              jax.ShapeDtypeStruct((NC, NP, 16), _f32)),
    mesh=_MESH,
    compiler_params=_SC_PARAMS,
    scratch_types=[
        pltpu.VMEM_SHARED((NP, 16), _f32),
        pltpu.VMEM_SHARED((NP, 16), _f32),
        pltpu.VMEM((MCH, 16), _f32),
        pltpu.VMEM((HIST_MAIN, MCH), jnp.int32),
        pltpu.VMEM((HIST_MAIN, MCH), jnp.int32),
        pltpu.VMEM((MCH,), jnp.int32),
        pltpu.VMEM((MCH,), jnp.int32),
        pltpu.SemaphoreType.DMA,
    ],
)


def _seg_body(t_hbm, e, zeros_hbm, out_hbm,
              acc, src_v, dst_v, xsrc_v, xdst_v, rows_v,
              g0, g1, g2, g3):
    cid = lax.axis_index("c")
    sid = lax.axis_index("s")
    w = cid * NS + sid
    r0 = sid * ROWS
    pltpu.sync_copy(zeros_hbm, acc.at[pl.ds(r0, ROWS)])
    pltpu.sync_copy(e.at[pl.ds(w * SEG_MAIN, SEG_MAIN), 0], src_v)
    pltpu.sync_copy(e.at[pl.ds(w * SEG_MAIN, SEG_MAIN), 1], dst_v)
    plsc.subcore_barrier()

    gs = (g0, g1, g2, g3)
    dummy = t_hbm.at[pl.ds(0, MCH)]
    for j in range(3):
        pltpu.async_copy(t_hbm.at[src_v.at[j]], rows_v.at[j], gs[j])

    @pl.loop(0, SEG_MAIN // 4 - 1)
    def _(g):
        c0 = g * 4
        for j in range(4):
            jn = (j + 3) % 4
            pltpu.make_async_copy(dummy, rows_v.at[j], gs[j]).wait()
            pltpu.async_copy(t_hbm.at[src_v.at[c0 + j + 3]], rows_v.at[jn],
                             gs[jn])
            pltpu.sync_copy(rows_v.at[j], acc.at[dst_v.at[c0 + j]], add=True)

    c0 = (SEG_MAIN // 4 - 1) * 4
    for j in range(6):
        c = c0 + j
        b = c % 4
        pltpu.make_async_copy(dummy, rows_v.at[b], gs[b]).wait()
        if c + 3 < SEG_MAIN:
            bn = (c + 3) % 4
            pltpu.async_copy(t_hbm.at[src_v.at[c + 3]], rows_v.at[bn], gs[bn])
        pltpu.sync_copy(rows_v.at[b], acc.at[dst_v.at[c]], add=True)

    @pl.when(w < XTRA)
    def _():
        pltpu.sync_copy(e.at[NT * SEG_MAIN + w, 0], xsrc_v)
        pltpu.sync_copy(e.at[NT * SEG_MAIN + w, 1], xdst_v)
        pltpu.async_copy(t_hbm.at[xsrc_v], rows_v.at[0], gs[0]).wait()
        pltpu.sync_copy(rows_v.at[0], acc.at[xdst_v], add=True)

    plsc.subcore_barrier()
    pltpu.sync_copy(acc.at[pl.ds(r0, ROWS)], out_hbm.at[cid, pl.ds(r0, ROWS)])


def _make_seg(h):
    return pl.kernel(
        _seg_body,
        out_type=jax.ShapeDtypeStruct((NC, NP, h), _f32),
        mesh=_MESH,
        compiler_params=_SC_PARAMS,
        scratch_types=[
            pltpu.VMEM_SHARED((NP, h), _f32),
            pltpu.VMEM((SEG_MAIN, MCH), jnp.int32),
            pltpu.VMEM((SEG_MAIN, MCH), jnp.int32),
            pltpu.VMEM((MCH,), jnp.int32),
            pltpu.VMEM((MCH,), jnp.int32),
            pltpu.VMEM((4, MCH, h), _f32),
            pltpu.SemaphoreType.DMA,
            pltpu.SemaphoreType.DMA,
            pltpu.SemaphoreType.DMA,
            pltpu.SemaphoreType.DMA,
        ],
    )


_seg = {h: _make_seg(h) for h in (H1, H2, H4)}


BT = 2048


def _pack128(y):
    r, h = y.shape
    k = 128 // h
    y3 = y.reshape(r // k, k, h)
    return jnp.concatenate([y3[:, j, :] for j in range(k)], axis=1)


def _l1_body(x_ref, hq_ref, w_ref, o_ref):
    p = lax.rsqrt(hq_ref[0, :, 0:1] + 1.0)
    t = jnp.dot(x_ref[...], w_ref[...], preferred_element_type=_f32) * p
    o_ref[...] = _pack128(t)


def _mid_body(parts_ref, hq_ref, hp_ref, w_ref, o_ref):
    a = parts_ref[0] + parts_ref[1]
    q = lax.rsqrt(hq_ref[0, :, 0:1] + 1.0)
    h = jnp.maximum(a * q, 0.0)
    p = lax.rsqrt(hp_ref[0, :, 0:1] + 1.0)
    t = jnp.dot(h, w_ref[...], preferred_element_type=_f32) * p
    o_ref[...] = _pack128(t)


def _fin_body(parts_ref, hq_ref, o_ref):
    q = lax.rsqrt(hq_ref[0, :, 0:1] + 1.0)
    o_ref[...] = (parts_ref[0] + parts_ref[1]) * q


def _hist_spec(set_idx):
    return pl.BlockSpec((1, BT, 16), lambda i, s=set_idx: (s, i, 0))


def _tc_l1(x, douts, w):
    din, dout = w.shape
    return pl.pallas_call(
        _l1_body,
        grid=(pl.cdiv(N, BT),),
        in_specs=[
            pl.BlockSpec((BT, din), lambda i: (i, 0)),
            _hist_spec(0),
            pl.BlockSpec((din, dout), lambda i: (0, 0)),
        ],
        out_specs=pl.BlockSpec((BT * dout // 128, 128), lambda i: (i, 0)),
        out_shape=jax.ShapeDtypeStruct((N * dout // 128, 128), _f32),
    )(x, douts, w)


def _tc_mid(parts, dins_a, douts_a, w, qset, pset):
    din, dout = w.shape
    return pl.pallas_call(
        _mid_body,
        grid=(pl.cdiv(N, BT),),
        in_specs=[
            pl.BlockSpec((NC, BT, din), lambda i: (0, i, 0)),
            _hist_spec(qset),
            _hist_spec(pset),
            pl.BlockSpec((din, dout), lambda i: (0, 0)),
        ],
        out_specs=pl.BlockSpec((BT * dout // 128, 128), lambda i: (i, 0)),
        out_shape=jax.ShapeDtypeStruct((N * dout // 128, 128), _f32),
    )(parts, dins_a, douts_a, w)


def _tc_fin(parts, dins_a, qset):
    dout = parts.shape[-1]
    return pl.pallas_call(
        _fin_body,
        grid=(pl.cdiv(N, BT),),
        in_specs=[
            pl.BlockSpec((NC, BT, dout), lambda i: (0, i, 0)),
            _hist_spec(qset),
        ],
        out_specs=pl.BlockSpec((BT, dout), lambda i: (i, 0)),
        out_shape=jax.ShapeDtypeStruct((N, dout), _f32),
    )(parts, dins_a)


def kernel(x, edge_index1, edge_index2, W1, W2, W3, W4):
    e1 = jnp.transpose(edge_index1.reshape(2, NCHK, MCH), (1, 0, 2))
    e2 = jnp.transpose(edge_index2.reshape(2, NCHK, MCH), (1, 0, 2))
    ones16 = jnp.ones((MCH, 16), _f32)
    z16 = jnp.zeros((ROWS, 16), _f32)
    douts, dins = _hist(e1, e2, ones16, z16)
    z64 = jnp.zeros((ROWS, H1), _f32)
    z32 = jnp.zeros((ROWS, H2), _f32)
    zh4 = jnp.zeros((ROWS, H4), _f32)

    t0 = _tc_l1(x, douts, W1)
    a1 = _seg[H1](t0.reshape(N, H1), e1, z64)
    t1 = _tc_mid(a1, dins, douts, W2, 0, 0)
    a2 = _seg[H2](t1.reshape(N, H2), e1, z32)
    t2 = _tc_mid(a2, dins, douts, W3, 0, 1)
    a3 = _seg[H3](t2.reshape(N, H3), e2, z32)
    t3 = _tc_mid(a3, dins, douts, W4, 1, 1)
    a4 = _seg[H4](t3.reshape(N, H4), e2, zh4)
    return _tc_fin(a4, dins, 1)


# speedup vs baseline: 47.3818x; 1.0228x over previous
import functools

import jax
import jax.numpy as jnp
from jax import lax
from jax.experimental import pallas as pl
from jax.experimental.pallas import tpu as pltpu
from jax.experimental.pallas import tpu_sc as plsc

N = 10000
E = 320000
D = 128
H1, H2, H3, H4 = 64, 32, 32, 16

NC, NS = 2, 16
NT = NC * NS
MCH = 128
NCHK = E // MCH
SEG_MAIN = NCHK // NT
HIST_MAIN = NCHK // NS
XTRA = NCHK - NT * SEG_MAIN
NP = 10240
ROWS = NP // NS

_MESH = plsc.VectorSubcoreMesh(
    core_axis_name="c", subcore_axis_name="s", num_cores=NC, num_subcores=NS)

_f32 = jnp.float32
_SC_PARAMS = pltpu.CompilerParams(use_tc_tiling_on_sc=False)


def _hist_body(e1, e2, ones_hbm, zeros_hbm, douts, dins,
               acc_do, acc_di, ones_v, src_v, dst_v, xsrc_v, xdst_v, hsem):
    cid = lax.axis_index("c")
    sid = lax.axis_index("s")
    r0 = sid * ROWS
    pltpu.sync_copy(ones_hbm, ones_v)
    pltpu.sync_copy(zeros_hbm, acc_do.at[pl.ds(r0, ROWS)])
    pltpu.sync_copy(zeros_hbm, acc_di.at[pl.ds(r0, ROWS)])
    plsc.subcore_barrier()

    def run(e):
        pltpu.sync_copy(e.at[pl.ds(sid * HIST_MAIN, HIST_MAIN), 0], src_v)
        pltpu.sync_copy(e.at[pl.ds(sid * HIST_MAIN, HIST_MAIN), 1], dst_v)

        for c in range(3):
            pltpu.async_copy(ones_v, acc_do.at[src_v.at[c]], hsem, add=True)
            pltpu.async_copy(ones_v, acc_di.at[dst_v.at[c]], hsem, add=True)

        @pl.loop(3, HIST_MAIN)
        def _(c):
            pltpu.async_copy(ones_v, acc_do.at[src_v.at[c]], hsem, add=True)
            pltpu.async_copy(ones_v, acc_di.at[dst_v.at[c]], hsem, add=True)
            pltpu.make_async_copy(ones_hbm, ones_v, hsem).wait()
            pltpu.make_async_copy(ones_hbm, ones_v, hsem).wait()

        for _c in range(3):
            pltpu.make_async_copy(ones_hbm, ones_v, hsem).wait()
            pltpu.make_async_copy(ones_hbm, ones_v, hsem).wait()

        @pl.when(sid < XTRA)
        def _():
            pltpu.sync_copy(e.at[NS * HIST_MAIN + sid, 0], xsrc_v)
            pltpu.sync_copy(e.at[NS * HIST_MAIN + sid, 1], xdst_v)
            d1 = pltpu.async_copy(ones_v, acc_do.at[xsrc_v], hsem, add=True)
            d2 = pltpu.async_copy(ones_v, acc_di.at[xdst_v], hsem, add=True)
            d1.wait()
            d2.wait()

    @pl.when(cid == 0)
    def _():
        run(e1)

    @pl.when(cid == 1)
    def _():
        run(e2)

    plsc.subcore_barrier()
    pltpu.sync_copy(acc_do.at[pl.ds(r0, ROWS)], douts.at[cid, pl.ds(r0, ROWS)])
    pltpu.sync_copy(acc_di.at[pl.ds(r0, ROWS)], dins.at[cid, pl.ds(r0, ROWS)])


_hist = pl.kernel(
    _hist_body,
    out_type=(jax.ShapeDtypeStruct((NC, NP, 16), _f32),
              jax.ShapeDtypeStruct((NC, NP, 16), _f32)),
    mesh=_MESH,
    compiler_params=_SC_PARAMS,
    scratch_types=[
        pltpu.VMEM_SHARED((NP, 16), _f32),
        pltpu.VMEM_SHARED((NP, 16), _f32),
        pltpu.VMEM((MCH, 16), _f32),
        pltpu.VMEM((HIST_MAIN, MCH), jnp.int32),
        pltpu.VMEM((HIST_MAIN, MCH), jnp.int32),
        pltpu.VMEM((MCH,), jnp.int32),
        pltpu.VMEM((MCH,), jnp.int32),
        pltpu.SemaphoreType.DMA,
    ],
)


def _seg_body(t_hbm, e, zeros_hbm, out_hbm,
              acc, src_v, dst_v, xsrc_v, xdst_v, rows_v,
              g0, g1, g2, g3):
    cid = lax.axis_index("c")
    sid = lax.axis_index("s")
    w = cid * NS + sid
    r0 = sid * ROWS
    pltpu.sync_copy(zeros_hbm, acc.at[pl.ds(r0, ROWS)])
    pltpu.sync_copy(e.at[pl.ds(w * SEG_MAIN, SEG_MAIN), 0], src_v)
    pltpu.sync_copy(e.at[pl.ds(w * SEG_MAIN, SEG_MAIN), 1], dst_v)
    plsc.subcore_barrier()

    gs = (g0, g1, g2, g3)
    dummy = t_hbm.at[pl.ds(0, MCH)]
    for j in range(3):
        pltpu.async_copy(t_hbm.at[src_v.at[j]], rows_v.at[j], gs[j])

    @pl.loop(0, SEG_MAIN // 4 - 1)
    def _(g):
        c0 = g * 4
        for j in range(4):
            jn = (j + 3) % 4
            pltpu.make_async_copy(dummy, rows_v.at[j], gs[j]).wait()
            pltpu.async_copy(t_hbm.at[src_v.at[c0 + j + 3]], rows_v.at[jn],
                             gs[jn])
            pltpu.sync_copy(rows_v.at[j], acc.at[dst_v.at[c0 + j]], add=True)

    c0 = (SEG_MAIN // 4 - 1) * 4
    for j in range(6):
        c = c0 + j
        b = c % 4
        pltpu.make_async_copy(dummy, rows_v.at[b], gs[b]).wait()
        if c + 3 < SEG_MAIN:
            bn = (c + 3) % 4
            pltpu.async_copy(t_hbm.at[src_v.at[c + 3]], rows_v.at[bn], gs[bn])
        pltpu.sync_copy(rows_v.at[b], acc.at[dst_v.at[c]], add=True)

    @pl.when(w < XTRA)
    def _():
        pltpu.sync_copy(e.at[NT * SEG_MAIN + w, 0], xsrc_v)
        pltpu.sync_copy(e.at[NT * SEG_MAIN + w, 1], xdst_v)
        pltpu.async_copy(t_hbm.at[xsrc_v], rows_v.at[0], gs[0]).wait()
        pltpu.sync_copy(rows_v.at[0], acc.at[xdst_v], add=True)

    plsc.subcore_barrier()
    pltpu.sync_copy(acc.at[pl.ds(r0, ROWS)], out_hbm.at[cid, pl.ds(r0, ROWS)])


def _make_seg(h):
    return pl.kernel(
        _seg_body,
        out_type=jax.ShapeDtypeStruct((NC, NP, h), _f32),
        mesh=_MESH,
        compiler_params=_SC_PARAMS,
        scratch_types=[
            pltpu.VMEM_SHARED((NP, h), _f32),
            pltpu.VMEM((SEG_MAIN, MCH), jnp.int32),
            pltpu.VMEM((SEG_MAIN, MCH), jnp.int32),
            pltpu.VMEM((MCH,), jnp.int32),
            pltpu.VMEM((MCH,), jnp.int32),
            pltpu.VMEM((4, MCH, h), _f32),
            pltpu.SemaphoreType.DMA,
            pltpu.SemaphoreType.DMA,
            pltpu.SemaphoreType.DMA,
            pltpu.SemaphoreType.DMA,
        ],
    )


_seg = {h: _make_seg(h) for h in (H1, H2, H4)}


BT = 2048


def _unpack128(a, h):
    r = a.shape[0]
    k = 128 // h
    stacked = jnp.stack([a[:, j * h:(j + 1) * h] for j in range(k)], axis=1)
    return stacked.reshape(r * k, h)


def _pack128(y):
    r, h = y.shape
    k = 128 // h
    y3 = y.reshape(r // k, k, h)
    return jnp.concatenate([y3[:, j, :] for j in range(k)], axis=1)


def _l1_body(x_ref, hq_ref, w_ref, o_ref):
    p = lax.rsqrt(hq_ref[0, :, 0:1] + 1.0)
    t = jnp.dot(x_ref[...], w_ref[...], preferred_element_type=_f32) * p
    o_ref[...] = _pack128(t)


def _mid_body(din, parts_ref, hq_ref, hp_ref, w_ref, o_ref):
    a = _unpack128(parts_ref[0] + parts_ref[1], din)
    q = lax.rsqrt(hq_ref[0, :, 0:1] + 1.0)
    h = jnp.maximum(a * q, 0.0)
    p = lax.rsqrt(hp_ref[0, :, 0:1] + 1.0)
    t = jnp.dot(h, w_ref[...], preferred_element_type=_f32) * p
    o_ref[...] = _pack128(t)


def _fin_body(parts_ref, hq_ref, o_ref):
    a = _unpack128(parts_ref[0] + parts_ref[1], 16)
    q = lax.rsqrt(hq_ref[0, :, 0:1] + 1.0)
    o_ref[...] = a * q


def _hist_spec(set_idx):
    return pl.BlockSpec((1, BT, 16), lambda i, s=set_idx: (s, i, 0))


def _tc_l1(x, douts, w):
    din, dout = w.shape
    return pl.pallas_call(
        _l1_body,
        grid=(pl.cdiv(N, BT),),
        in_specs=[
            pl.BlockSpec((BT, din), lambda i: (i, 0)),
            _hist_spec(0),
            pl.BlockSpec((din, dout), lambda i: (0, 0)),
        ],
        out_specs=pl.BlockSpec((BT * dout // 128, 128), lambda i: (i, 0)),
        out_shape=jax.ShapeDtypeStruct((N * dout // 128, 128), _f32),
    )(x, douts, w)


def _tc_mid(parts, dins_a, douts_a, w, qset, pset):
    din, dout = w.shape
    return pl.pallas_call(
        functools.partial(_mid_body, din),
        grid=(pl.cdiv(N, BT),),
        in_specs=[
            pl.BlockSpec((NC, BT * din // 128, 128), lambda i: (0, i, 0)),
            _hist_spec(qset),
            _hist_spec(pset),
            pl.BlockSpec((din, dout), lambda i: (0, 0)),
        ],
        out_specs=pl.BlockSpec((BT * dout // 128, 128), lambda i: (i, 0)),
        out_shape=jax.ShapeDtypeStruct((N * dout // 128, 128), _f32),
    )(parts, dins_a, douts_a, w)


def _tc_fin(parts, dins_a, qset):
    return pl.pallas_call(
        _fin_body,
        grid=(pl.cdiv(N, BT),),
        in_specs=[
            pl.BlockSpec((NC, BT * 16 // 128, 128), lambda i: (0, i, 0)),
            _hist_spec(qset),
        ],
        out_specs=pl.BlockSpec((BT, 16), lambda i: (i, 0)),
        out_shape=jax.ShapeDtypeStruct((N, 16), _f32),
    )(parts, dins_a)


def kernel(x, edge_index1, edge_index2, W1, W2, W3, W4):
    e1 = jnp.transpose(edge_index1.reshape(2, NCHK, MCH), (1, 0, 2))
    e2 = jnp.transpose(edge_index2.reshape(2, NCHK, MCH), (1, 0, 2))
    ones16 = jnp.ones((MCH, 16), _f32)
    z16 = jnp.zeros((ROWS, 16), _f32)
    douts, dins = _hist(e1, e2, ones16, z16)
    z64 = jnp.zeros((ROWS, H1), _f32)
    z32 = jnp.zeros((ROWS, H2), _f32)
    zh4 = jnp.zeros((ROWS, H4), _f32)

    def pk(a, h):
        return a.reshape(NC, NP * h // 128, 128)

    t0 = _tc_l1(x, douts, W1)
    a1 = _seg[H1](t0.reshape(N, H1), e1, z64)
    t1 = _tc_mid(pk(a1, H1), dins, douts, W2, 0, 0)
    a2 = _seg[H2](t1.reshape(N, H2), e1, z32)
    t2 = _tc_mid(pk(a2, H2), dins, douts, W3, 0, 1)
    a3 = _seg[H3](t2.reshape(N, H3), e2, z32)
    t3 = _tc_mid(pk(a3, H3), dins, douts, W4, 1, 1)
    a4 = _seg[H4](t3.reshape(N, H4), e2, zh4)
    return _tc_fin(pk(a4, H4), dins, 1)


# speedup vs baseline: 49.0060x; 1.0343x over previous
import functools

import jax
import jax.numpy as jnp
from jax import lax
from jax.experimental import pallas as pl
from jax.experimental.pallas import tpu as pltpu
from jax.experimental.pallas import tpu_sc as plsc

N = 10000
E = 320000
D = 128
H1, H2, H3, H4 = 64, 32, 32, 16

NC, NS = 2, 16
NT = NC * NS
MCH = 128
NCHK = E // MCH
SEG_MAIN = NCHK // NT
HIST_MAIN = NCHK // NS
XTRA = NCHK - NT * SEG_MAIN
NP = 10240
ROWS = NP // NS

_MESH = plsc.VectorSubcoreMesh(
    core_axis_name="c", subcore_axis_name="s", num_cores=NC, num_subcores=NS)

_f32 = jnp.float32
_SC_PARAMS = pltpu.CompilerParams(use_tc_tiling_on_sc=False)


def _hist_body(e1, e2, ones_hbm, zeros_hbm, douts, dins,
               acc_do, acc_di, ones_v, src_v, dst_v, xsrc_v, xdst_v, hsem):
    cid = lax.axis_index("c")
    sid = lax.axis_index("s")
    r0 = sid * ROWS
    pltpu.sync_copy(ones_hbm, ones_v)
    pltpu.sync_copy(zeros_hbm, acc_do.at[pl.ds(r0, ROWS)])
    pltpu.sync_copy(zeros_hbm, acc_di.at[pl.ds(r0, ROWS)])
    plsc.subcore_barrier()

    def run(e):
        pltpu.sync_copy(e.at[pl.ds(sid * HIST_MAIN, HIST_MAIN), 0], src_v)
        pltpu.sync_copy(e.at[pl.ds(sid * HIST_MAIN, HIST_MAIN), 1], dst_v)

        for c in range(3):
            pltpu.async_copy(ones_v, acc_do.at[src_v.at[c]], hsem, add=True)
            pltpu.async_copy(ones_v, acc_di.at[dst_v.at[c]], hsem, add=True)

        @pl.loop(3, HIST_MAIN)
        def _(c):
            pltpu.async_copy(ones_v, acc_do.at[src_v.at[c]], hsem, add=True)
            pltpu.async_copy(ones_v, acc_di.at[dst_v.at[c]], hsem, add=True)
            pltpu.make_async_copy(ones_hbm, ones_v, hsem).wait()
            pltpu.make_async_copy(ones_hbm, ones_v, hsem).wait()

        for _c in range(3):
            pltpu.make_async_copy(ones_hbm, ones_v, hsem).wait()
            pltpu.make_async_copy(ones_hbm, ones_v, hsem).wait()

        @pl.when(sid < XTRA)
        def _():
            pltpu.sync_copy(e.at[NS * HIST_MAIN + sid, 0], xsrc_v)
            pltpu.sync_copy(e.at[NS * HIST_MAIN + sid, 1], xdst_v)
            d1 = pltpu.async_copy(ones_v, acc_do.at[xsrc_v], hsem, add=True)
            d2 = pltpu.async_copy(ones_v, acc_di.at[xdst_v], hsem, add=True)
            d1.wait()
            d2.wait()

    @pl.when(cid == 0)
    def _():
        run(e1)

    @pl.when(cid == 1)
    def _():
        run(e2)

    plsc.subcore_barrier()
    pltpu.sync_copy(acc_do.at[pl.ds(r0, ROWS)], douts.at[cid, pl.ds(r0, ROWS)])
    pltpu.sync_copy(acc_di.at[pl.ds(r0, ROWS)], dins.at[cid, pl.ds(r0, ROWS)])


_hist = pl.kernel(
    _hist_body,
    out_type=(jax.ShapeDtypeStruct((NC, NP, 16), _f32),
              jax.ShapeDtypeStruct((NC, NP, 16), _f32)),
    mesh=_MESH,
    compiler_params=_SC_PARAMS,
    scratch_types=[
        pltpu.VMEM_SHARED((NP, 16), _f32),
        pltpu.VMEM_SHARED((NP, 16), _f32),
        pltpu.VMEM((MCH, 16), _f32),
        pltpu.VMEM((HIST_MAIN, MCH), jnp.int32),
        pltpu.VMEM((HIST_MAIN, MCH), jnp.int32),
        pltpu.VMEM((MCH,), jnp.int32),
        pltpu.VMEM((MCH,), jnp.int32),
        pltpu.SemaphoreType.DMA,
    ],
)


def _seg_body(t_hbm, e, zeros_hbm, out_hbm,
              acc, src_v, dst_v, xsrc_v, xdst_v, rows_v,
              g0, g1, g2, g3):
    cid = lax.axis_index("c")
    sid = lax.axis_index("s")
    w = cid * NS + sid
    r0 = sid * ROWS
    pltpu.sync_copy(zeros_hbm, acc.at[pl.ds(r0, ROWS)])
    pltpu.sync_copy(e.at[pl.ds(w * SEG_MAIN, SEG_MAIN), 0], src_v)
    pltpu.sync_copy(e.at[pl.ds(w * SEG_MAIN, SEG_MAIN), 1], dst_v)
    plsc.subcore_barrier()

    gs = (g0, g1, g2, g3)
    dummy = t_hbm.at[pl.ds(0, MCH)]
    for j in range(3):
        pltpu.async_copy(t_hbm.at[src_v.at[j]], rows_v.at[j], gs[j])

    @pl.loop(0, SEG_MAIN // 4 - 1)
    def _(g):
        c0 = g * 4
        for j in range(4):
            jn = (j + 3) % 4
            pltpu.make_async_copy(dummy, rows_v.at[j], gs[j]).wait()
            pltpu.async_copy(t_hbm.at[src_v.at[c0 + j + 3]], rows_v.at[jn],
                             gs[jn])
            pltpu.sync_copy(rows_v.at[j], acc.at[dst_v.at[c0 + j]], add=True)

    c0 = (SEG_MAIN // 4 - 1) * 4
    for j in range(6):
        c = c0 + j
        b = c % 4
        pltpu.make_async_copy(dummy, rows_v.at[b], gs[b]).wait()
        if c + 3 < SEG_MAIN:
            bn = (c + 3) % 4
            pltpu.async_copy(t_hbm.at[src_v.at[c + 3]], rows_v.at[bn], gs[bn])
        pltpu.sync_copy(rows_v.at[b], acc.at[dst_v.at[c]], add=True)

    @pl.when(w < XTRA)
    def _():
        pltpu.sync_copy(e.at[NT * SEG_MAIN + w, 0], xsrc_v)
        pltpu.sync_copy(e.at[NT * SEG_MAIN + w, 1], xdst_v)
        pltpu.async_copy(t_hbm.at[xsrc_v], rows_v.at[0], gs[0]).wait()
        pltpu.sync_copy(rows_v.at[0], acc.at[xdst_v], add=True)

    plsc.subcore_barrier()
    pltpu.sync_copy(acc.at[pl.ds(r0, ROWS)], out_hbm.at[cid, pl.ds(r0, ROWS)])


def _make_seg(h):
    return pl.kernel(
        _seg_body,
        out_type=jax.ShapeDtypeStruct((NC, NP, h), _f32),
        mesh=_MESH,
        compiler_params=_SC_PARAMS,
        scratch_types=[
            pltpu.VMEM_SHARED((NP, h), _f32),
            pltpu.VMEM((SEG_MAIN, MCH), jnp.int32),
            pltpu.VMEM((SEG_MAIN, MCH), jnp.int32),
            pltpu.VMEM((MCH,), jnp.int32),
            pltpu.VMEM((MCH,), jnp.int32),
            pltpu.VMEM((4, MCH, h), _f32),
            pltpu.SemaphoreType.DMA,
            pltpu.SemaphoreType.DMA,
            pltpu.SemaphoreType.DMA,
            pltpu.SemaphoreType.DMA,
        ],
    )


_seg = {h: _make_seg(h) for h in (H1, H2, H4)}


BT = 2048


def _unpack128(a, h):
    r = a.shape[0]
    k = 128 // h
    stacked = jnp.stack([a[:, j * h:(j + 1) * h] for j in range(k)], axis=1)
    return stacked.reshape(r * k, h)


def _pack128(y):
    r, h = y.shape
    k = 128 // h
    y3 = y.reshape(r // k, k, h)
    return jnp.concatenate([y3[:, j, :] for j in range(k)], axis=1)


def _l1_body(x_ref, hq_ref, w_ref, o_ref):
    p = lax.rsqrt(hq_ref[0, :, 0:1] + 1.0)
    t = jnp.dot(x_ref[...], w_ref[...], preferred_element_type=_f32) * p
    o_ref[...] = _pack128(t)


def _mid_body(din, parts_ref, hq_ref, hp_ref, w_ref, o_ref):
    a = _unpack128(parts_ref[0] + parts_ref[1], din)
    h = jnp.maximum(a, 0.0)
    s = lax.rsqrt((hq_ref[0, :, 0:1] + 1.0) * (hp_ref[0, :, 0:1] + 1.0))
    t = jnp.dot(h, w_ref[...], preferred_element_type=_f32) * s
    o_ref[...] = _pack128(t)


def _fin_body(parts_ref, hq_ref, o_ref):
    a = _unpack128(parts_ref[0] + parts_ref[1], 16)
    q = lax.rsqrt(hq_ref[0, :, 0:1] + 1.0)
    o_ref[...] = a * q


def _hist_spec(set_idx):
    return pl.BlockSpec((1, BT, 16), lambda i, s=set_idx: (s, i, 0))


def _tc_l1(x, douts, w):
    din, dout = w.shape
    return pl.pallas_call(
        _l1_body,
        grid=(pl.cdiv(N, BT),),
        in_specs=[
            pl.BlockSpec((BT, din), lambda i: (i, 0)),
            _hist_spec(0),
            pl.BlockSpec((din, dout), lambda i: (0, 0)),
        ],
        out_specs=pl.BlockSpec((BT * dout // 128, 128), lambda i: (i, 0)),
        out_shape=jax.ShapeDtypeStruct((N * dout // 128, 128), _f32),
    )(x, douts, w)


def _tc_mid(parts, dins_a, douts_a, w, qset, pset):
    din, dout = w.shape
    return pl.pallas_call(
        functools.partial(_mid_body, din),
        grid=(pl.cdiv(N, BT),),
        in_specs=[
            pl.BlockSpec((NC, BT * din // 128, 128), lambda i: (0, i, 0)),
            _hist_spec(qset),
            _hist_spec(pset),
            pl.BlockSpec((din, dout), lambda i: (0, 0)),
        ],
        out_specs=pl.BlockSpec((BT * dout // 128, 128), lambda i: (i, 0)),
        out_shape=jax.ShapeDtypeStruct((N * dout // 128, 128), _f32),
    )(parts, dins_a, douts_a, w)


def _tc_fin(parts, dins_a, qset):
    return pl.pallas_call(
        _fin_body,
        grid=(pl.cdiv(N, BT),),
        in_specs=[
            pl.BlockSpec((NC, BT * 16 // 128, 128), lambda i: (0, i, 0)),
            _hist_spec(qset),
        ],
        out_specs=pl.BlockSpec((BT, 16), lambda i: (i, 0)),
        out_shape=jax.ShapeDtypeStruct((N, 16), _f32),
    )(parts, dins_a)


def kernel(x, edge_index1, edge_index2, W1, W2, W3, W4):
    e1 = jnp.transpose(edge_index1.reshape(2, NCHK, MCH), (1, 0, 2))
    e2 = jnp.transpose(edge_index2.reshape(2, NCHK, MCH), (1, 0, 2))
    ones16 = jnp.ones((MCH, 16), _f32)
    z16 = jnp.zeros((ROWS, 16), _f32)
    douts, dins = _hist(e1, e2, ones16, z16)
    z64 = jnp.zeros((ROWS, H1), _f32)
    z32 = jnp.zeros((ROWS, H2), _f32)
    zh4 = jnp.zeros((ROWS, H4), _f32)

    def pk(a, h):
        return a.reshape(NC, NP * h // 128, 128)

    t0 = _tc_l1(x, douts, W1)
    a1 = _seg[H1](t0.reshape(N, H1), e1, z64)
    t1 = _tc_mid(pk(a1, H1), dins, douts, W2, 0, 0)
    a2 = _seg[H2](t1.reshape(N, H2), e1, z32)
    t2 = _tc_mid(pk(a2, H2), dins, douts, W3, 0, 1)
    a3 = _seg[H3](t2.reshape(N, H3), e2, z32)
    t3 = _tc_mid(pk(a3, H3), dins, douts, W4, 1, 1)
    a4 = _seg[H4](t3.reshape(N, H4), e2, zh4)
    return _tc_fin(pk(a4, H4), dins, 1)
